# Initial kernel scaffold; baseline (speedup 1.0000x reference)
import jax
import jax.numpy as jnp
from jax.experimental import pallas as pl


def kernel(heads, rels, tails, entity_emb, rel_emb):
    raise NotImplementedError("write your pallas kernel here")


# speedup vs baseline: 2.1962x; 2.1962x over previous
import dataclasses
import functools

import jax
import jax.numpy as jnp
from jax import lax
from jax.experimental import pallas as pl
from jax.experimental.pallas import tpu as pltpu
from jax.experimental.pallas import tpu_sc as plsc

NC = 2
NS = 16
L = 16
NW = NC * NS

B = 16384
D = 128
BPW = B // NW
CH = 128
NCH = BPW // CH


@functools.cache
def _mesh():
    return plsc.VectorSubcoreMesh(
        core_axis_name="c", subcore_axis_name="s", num_cores=NC, num_subcores=NS
    )


@functools.cache
def _compiler_params():
    cp = pltpu.CompilerParams()
    if "needs_layout_passes" in pltpu.CompilerParams.__dataclass_fields__:
        cp = dataclasses.replace(cp, needs_layout_passes=False)
    return cp


def _distmult_body(heads_hbm, rels_hbm, tails_hbm, ent_hbm, rel_hbm, out_hbm,
                   hidx, ridx, tidx, hbufs, rbufs, tbufs, tr, scores, sems):
    wid = lax.axis_index("s") * NC + lax.axis_index("c")
    base = wid * BPW

    pltpu.sync_copy(heads_hbm.at[pl.ds(base, BPW)], hidx)
    pltpu.sync_copy(rels_hbm.at[pl.ds(base, BPW)], ridx)
    pltpu.sync_copy(tails_hbm.at[pl.ds(base, BPW)], tidx)

    iota = lax.iota(jnp.int32, L)

    def issue(g, slot):
        c0 = g * CH
        return (
            pltpu.async_copy(ent_hbm.at[hidx.at[pl.ds(c0, CH)]],
                             hbufs.at[slot], sems.at[slot, 0]),
            pltpu.async_copy(rel_hbm.at[ridx.at[pl.ds(c0, CH)]],
                             rbufs.at[slot], sems.at[slot, 1]),
            pltpu.async_copy(ent_hbm.at[tidx.at[pl.ds(c0, CH)]],
                             tbufs.at[slot], sems.at[slot, 2]),
        )

    def compute(g, slot):
        hb, rb, tb = hbufs.at[slot], rbufs.at[slot], tbufs.at[slot]
        c0 = g * CH

        @pl.loop(0, CH, step=L)
        def _block(w0):
            for wi in range(L):
                w = w0 + wi
                acc = hb[w, 0:L] * rb[w, 0:L] * tb[w, 0:L]
                for cc in range(1, D // L):
                    sl = pl.ds(cc * L, L)
                    acc = acc + hb[w, sl] * rb[w, sl] * tb[w, sl]
                tr[wi, :] = acc
            s = plsc.load_gather(tr, [iota, jnp.zeros((L,), jnp.int32)])
            for cc in range(1, L):
                s = s + plsc.load_gather(tr, [iota, jnp.full((L,), cc, jnp.int32)])
            scores[pl.ds(c0 + w0, L)] = s

    cps = issue(0, 0)
    for g in range(NCH):
        slot = g % 2
        for cp in cps:
            cp.wait()
        if g + 1 < NCH:
            cps = issue(g + 1, 1 - slot)
        compute(g, slot)

    pltpu.sync_copy(scores, out_hbm.at[pl.ds(base, BPW)])


@jax.jit
def _distmult(heads, rels, tails, entity_emb, rel_emb):
    run = pl.kernel(
        _distmult_body,
        out_type=jax.ShapeDtypeStruct((B,), jnp.float32),
        mesh=_mesh(),
        compiler_params=_compiler_params(),
        scratch_types=[
            pltpu.VMEM((BPW,), jnp.int32),
            pltpu.VMEM((BPW,), jnp.int32),
            pltpu.VMEM((BPW,), jnp.int32),
            pltpu.VMEM((2, CH, D), jnp.float32),
            pltpu.VMEM((2, CH, D), jnp.float32),
            pltpu.VMEM((2, CH, D), jnp.float32),
            pltpu.VMEM((L, L), jnp.float32),
            pltpu.VMEM((BPW,), jnp.float32),
            pltpu.SemaphoreType.DMA((2, 3)),
        ],
    )
    return run(heads, rels, tails, entity_emb, rel_emb)


def kernel(heads, rels, tails, entity_emb, rel_emb):
    heads = heads.astype(jnp.int32)
    rels = rels.astype(jnp.int32)
    tails = tails.astype(jnp.int32)
    return _distmult(heads, rels, tails, entity_emb, rel_emb)
